# baseline (speedup 1.0000x reference)
import functools

import jax
import jax.numpy as jnp
from jax import lax
from jax.experimental import pallas as pl
from jax.experimental.pallas import tpu as pltpu
from jax.experimental.pallas import tpu_sc as plsc

N = 10000
E = 160000
VOCAB = 100000
EMB = 256
H = 128
NG = 64

NC, NS = 2, 16
NW = NC * NS

NPAD = 10240
NJUNK = N
NPA = 10112
RWA = NPA // NS

EPAD = 163840
CH = 128
CHUNKS = EPAD // CH
CHAL = 1344
NBUF = 2
N0C = 56
N1C = 24
assert 16 * (N0C + N1C) == CHUNKS
T = CHUNKS // NW

BPW = NPAD // NW
BCH = 40
BT = BPW // BCH
EBUF = 2

BLK = 1000
GRID = N // BLK

_mesh = plsc.VectorSubcoreMesh(
    core_axis_name="c", subcore_axis_name="s", num_cores=NC, num_subcores=NS)


@functools.partial(
    pl.kernel,
    out_type=jax.ShapeDtypeStruct((NPAD, EMB), jnp.float32),
    mesh=_mesh,
    scratch_types=[
        pltpu.VMEM((BT, BCH), jnp.int32),
        [pltpu.VMEM((BCH, EMB), jnp.float32)] * EBUF,
        [pltpu.SemaphoreType.DMA] * EBUF,
        [pltpu.SemaphoreType.DMA] * EBUF,
    ],
)
def _emb_gather(table_hbm, idx_hbm, out_hbm, idx_v, bufs, gsems, wsems):
    wid = lax.axis_index("s") * NC + lax.axis_index("c")
    base = wid * BPW
    pltpu.sync_copy(idx_hbm.at[wid], idx_v)

    def start_g(b, p):
        pltpu.async_copy(table_hbm.at[idx_v.at[b]], bufs[p], gsems[p])

    def out_at(b):
        return out_hbm.at[pl.ds(base + b * BCH, BCH)]

    for p in range(EBUF):
        start_g(p, p)
    for b in range(BT):
        p = b % EBUF
        pltpu.make_async_copy(table_hbm.at[idx_v.at[b]], bufs[p],
                              gsems[p]).wait()
        pltpu.async_copy(bufs[p], out_at(b), wsems[p])
        if b + EBUF < BT:
            pltpu.make_async_copy(bufs[p], out_at(b), wsems[p]).wait()
            start_g(b + EBUF, p)
    for b in range(BT - EBUF, BT):
        p = b % EBUF
        pltpu.make_async_copy(bufs[p], out_at(b), wsems[p]).wait()


@functools.partial(
    pl.kernel,
    out_type=jax.ShapeDtypeStruct((NC, NPA, 16), jnp.float32),
    mesh=_mesh,
    scratch_types=[
        pltpu.VMEM((T, CH), jnp.int32),
        pltpu.VMEM((CH, 16), jnp.float32),
        pltpu.VMEM_SHARED((NPA, 16), jnp.float32),
        pltpu.SemaphoreType.DMA,
    ],
)
def _degree(dst_hbm, zeros_hbm, ones_hbm, out_hbm, dst_v, ones_v, acc_sh, sem):
    cid = lax.axis_index("c")
    sid = lax.axis_index("s")
    wid = sid * NC + cid
    pltpu.sync_copy(zeros_hbm, acc_sh.at[pl.ds(sid * RWA, RWA)])
    pltpu.sync_copy(ones_hbm, ones_v)
    pltpu.sync_copy(dst_hbm.at[pl.ds(wid * T, T)], dst_v)
    plsc.subcore_barrier()

    @pl.loop(0, T)
    def _(t):
        pltpu.async_copy(ones_v, acc_sh.at[dst_v.at[t]], sem, add=True)

    @pl.loop(0, T)
    def _(t):
        pltpu.make_async_copy(ones_v, acc_sh.at[dst_v.at[t]], sem).wait()

    plsc.subcore_barrier()
    pltpu.sync_copy(acc_sh.at[pl.ds(sid * RWA, RWA)],
                    out_hbm.at[cid, pl.ds(sid * RWA, RWA), :])


@functools.partial(
    pl.kernel,
    out_type=jax.ShapeDtypeStruct((NC, NPA, H), jnp.float32),
    mesh=_mesh,
    scratch_types=[
        pltpu.VMEM((N0C, CH), jnp.int32),
        pltpu.VMEM((N0C, CH), jnp.int32),
        [pltpu.VMEM((CH, H), jnp.float32)] * NBUF,
        pltpu.VMEM_SHARED((NPA, H), jnp.float32),
        [pltpu.SemaphoreType.DMA] * NBUF,
        [pltpu.SemaphoreType.DMA] * NBUF,
    ],
)
def _edge_pass(g_hbm, src_hbm, dst_hbm, zeros_hbm, out_hbm,
               src_v, dst_v, bufs, acc_sh, gsems, ssems):
    cid = lax.axis_index("c")
    sid = lax.axis_index("s")
    pltpu.sync_copy(zeros_hbm, acc_sh.at[pl.ds(sid * RWA, RWA)])

    def wait_gather(t, b):
        pltpu.make_async_copy(g_hbm.at[src_v.at[t]], bufs[b], gsems[b]).wait()

    def start_scatter(t, b):
        pltpu.async_copy(bufs[b], acc_sh.at[dst_v.at[t]], ssems[b], add=True)

    def wait_scatter(t, b):
        pltpu.make_async_copy(bufs[b], acc_sh.at[dst_v.at[t]],
                              ssems[b]).wait()

    def prefetch(wstart):
        pltpu.sync_copy(src_hbm.at[pl.ds(wstart, N0C)], src_v)
        pltpu.sync_copy(dst_hbm.at[pl.ds(wstart, N0C)], dst_v)
        for b in range(NBUF):
            pltpu.async_copy(g_hbm.at[src_v.at[b]], bufs[b], gsems[b])

    def run(nchunks):
        @pl.loop(0, nchunks // NBUF - 1)
        def _(i):
            t0 = i * NBUF
            for b in range(NBUF):
                wait_gather(t0 + b, b)
                start_scatter(t0 + b, b)
            for b in range(NBUF):
                wait_scatter(t0 + b, b)
                pltpu.async_copy(g_hbm.at[src_v.at[t0 + NBUF + b]], bufs[b],
                                 gsems[b])

        t0 = nchunks - NBUF
        for b in range(NBUF):
            wait_gather(t0 + b, b)
            start_scatter(t0 + b, b)
        for b in range(NBUF):
            wait_scatter(t0 + b, b)

    @pl.when(cid == 1)
    def _():
        prefetch(sid * N0C)

    @pl.when(cid == 0)
    def _():
        prefetch(16 * N0C + sid * N1C)

    plsc.subcore_barrier()

    @pl.when(cid == 1)
    def _():
        run(N0C)

    @pl.when(cid == 0)
    def _():
        run(N1C)

    plsc.subcore_barrier()
    pltpu.sync_copy(acc_sh.at[pl.ds(sid * RWA, RWA)],
                    out_hbm.at[cid, pl.ds(sid * RWA, RWA), :])


def _tc_a_body(h0_ref, w1_ref, d0_ref, d1_ref, dinv_ref, g1_ref):
    deg = d0_ref[...] + d1_ref[...] + 1.0
    dinv = lax.rsqrt(deg)
    h1 = jnp.dot(h0_ref[...], w1_ref[...], preferred_element_type=jnp.float32)
    dinv_ref[...] = dinv
    g1_ref[...] = h1 * dinv


def _tc_b_body(s0_ref, s1_ref, g1_ref, dinv_ref,
               w2_ref, b1_ref, g2_ref):
    dinv = dinv_ref[...]
    s = s0_ref[...] + s1_ref[...]
    x = jnp.maximum((s + g1_ref[...]) * dinv + b1_ref[...], 0.0)
    g2_ref[...] = jnp.dot(
        x, w2_ref[...], preferred_element_type=jnp.float32) * dinv


def _tc_c_body(s0_ref, s1_ref, g2_ref, dinv_ref, b2_ref,
               batch_ref, wc_ref, bc_ref, out_ref, pool_acc, cnt_acc):
    i = pl.program_id(0)

    @pl.when(i == 0)
    def _():
        pool_acc[...] = jnp.zeros_like(pool_acc)
        cnt_acc[...] = jnp.zeros_like(cnt_acc)

    s = s0_ref[...] + s1_ref[...]
    x = jnp.maximum(
        (s + g2_ref[...]) * dinv_ref[...] + b2_ref[...], 0.0)
    gids = lax.broadcasted_iota(jnp.int32, (NG, BLK), 0)
    ind = (batch_ref[...].reshape(1, BLK) == gids).astype(jnp.float32)
    pool_acc[...] += jnp.dot(ind, x, preferred_element_type=jnp.float32)
    cnt_acc[...] += jnp.sum(ind, axis=1, keepdims=True)

    @pl.when(i == GRID - 1)
    def _():
        pooled = pool_acc[...] / jnp.maximum(cnt_acc[...], 1.0)
        out_ref[...] = jnp.dot(
            pooled, wc_ref[...], preferred_element_type=jnp.float32) \
            + bc_ref[...]


def _row_spec(width):
    return pl.BlockSpec((BLK, width), lambda i: (i, 0))


def _full_spec(shape):
    return pl.BlockSpec(shape, lambda i: tuple(0 for _ in shape))


def kernel(x, edge_index, batch, emb_table, W1, b1, W2, b2, Wc, bc):
    x = x.astype(jnp.int32)
    src = edge_index[0].astype(jnp.int32)
    dst = edge_index[1].astype(jnp.int32)
    batch = batch.astype(jnp.int32)

    xpad = jnp.concatenate([x, jnp.zeros((NPAD - N,), jnp.int32)])
    xpad = xpad.reshape(NW, BT, BCH)
    srcp = jnp.concatenate(
        [src, jnp.zeros((CHAL * CH - E,), jnp.int32)]).reshape(CHAL, CH)
    dstp = jnp.concatenate(
        [dst, jnp.full((CHAL * CH - E,), NJUNK, jnp.int32)]).reshape(CHAL, CH)

    zrow = jnp.zeros((RWA, H), jnp.float32)
    zdeg = jnp.zeros((RWA, 16), jnp.float32)
    ones16 = jnp.ones((CH, 16), jnp.float32)

    h0 = _emb_gather(emb_table, xpad)[:N]
    degp = _degree(dstp, zdeg, ones16)
    d0 = degp[0, :N, 0:1]
    d1 = degp[1, :N, 0:1]

    dinv, g1 = pl.pallas_call(
        _tc_a_body,
        grid=(GRID,),
        in_specs=[_row_spec(EMB), _full_spec((EMB, H)),
                  _row_spec(1), _row_spec(1)],
        out_specs=[_row_spec(1), _row_spec(H)],
        out_shape=[jax.ShapeDtypeStruct((N, 1), jnp.float32),
                   jax.ShapeDtypeStruct((N, H), jnp.float32)],
        compiler_params=pltpu.CompilerParams(
            dimension_semantics=("parallel",)),
    )(h0, W1, d0, d1)

    s1p = _edge_pass(g1, srcp, dstp, zrow)

    g2 = pl.pallas_call(
        _tc_b_body,
        grid=(GRID,),
        in_specs=[_row_spec(H), _row_spec(H), _row_spec(H), _row_spec(1),
                  _full_spec((H, H)), _full_spec((1, H))],
        out_specs=_row_spec(H),
        out_shape=jax.ShapeDtypeStruct((N, H), jnp.float32),
        compiler_params=pltpu.CompilerParams(
            dimension_semantics=("parallel",)),
    )(s1p[0, :N], s1p[1, :N], g1, dinv, W2, b1.reshape(1, H))

    s2p = _edge_pass(g2, srcp, dstp, zrow)

    out = pl.pallas_call(
        _tc_c_body,
        grid=(GRID,),
        in_specs=[_row_spec(H), _row_spec(H), _row_spec(H), _row_spec(1),
                  _full_spec((1, H)),
                  pl.BlockSpec((1, 1, BLK), lambda i: (i, 0, 0)),
                  _full_spec((H, 1)), _full_spec((1, 1))],
        out_specs=_full_spec((NG, 1)),
        out_shape=jax.ShapeDtypeStruct((NG, 1), jnp.float32),
        scratch_shapes=[pltpu.VMEM((NG, H), jnp.float32),
                        pltpu.VMEM((NG, 1), jnp.float32)],
        compiler_params=pltpu.CompilerParams(
            dimension_semantics=("arbitrary",)),
    )(s2p[0, :N], s2p[1, :N], g2, dinv, b2.reshape(1, H),
      batch.reshape(GRID, 1, BLK), Wc, bc.reshape(1, 1))

    return out


# speedup vs baseline: 1.0270x; 1.0270x over previous
import functools

import jax
import jax.numpy as jnp
from jax import lax
from jax.experimental import pallas as pl
from jax.experimental.pallas import tpu as pltpu
from jax.experimental.pallas import tpu_sc as plsc

N = 10000
E = 160000
VOCAB = 100000
EMB = 256
H = 128
NG = 64

NC, NS = 2, 16
NW = NC * NS

NPAD = 10240
NJUNK = N
NPA = 10112
RWA = NPA // NS

EPAD = 163840
CH = 128
CHUNKS = EPAD // CH
CHAL = 1344
NBUF = 2
N0C = 64
N1C = 16
PH = 32
assert 16 * (N0C + N1C) == CHUNKS
T = CHUNKS // NW

BPW = NPAD // NW
BCH = 40
BT = BPW // BCH
EBUF = 2

BLK = 1000
GRID = N // BLK

_mesh = plsc.VectorSubcoreMesh(
    core_axis_name="c", subcore_axis_name="s", num_cores=NC, num_subcores=NS)


@functools.partial(
    pl.kernel,
    out_type=jax.ShapeDtypeStruct((NPAD, EMB), jnp.float32),
    mesh=_mesh,
    scratch_types=[
        pltpu.VMEM((BT, BCH), jnp.int32),
        [pltpu.VMEM((BCH, EMB), jnp.float32)] * EBUF,
        [pltpu.SemaphoreType.DMA] * EBUF,
        [pltpu.SemaphoreType.DMA] * EBUF,
    ],
)
def _emb_gather(table_hbm, idx_hbm, out_hbm, idx_v, bufs, gsems, wsems):
    wid = lax.axis_index("s") * NC + lax.axis_index("c")
    base = wid * BPW
    pltpu.sync_copy(idx_hbm.at[wid], idx_v)

    def start_g(b, p):
        pltpu.async_copy(table_hbm.at[idx_v.at[b]], bufs[p], gsems[p])

    def out_at(b):
        return out_hbm.at[pl.ds(base + b * BCH, BCH)]

    for p in range(EBUF):
        start_g(p, p)
    for b in range(BT):
        p = b % EBUF
        pltpu.make_async_copy(table_hbm.at[idx_v.at[b]], bufs[p],
                              gsems[p]).wait()
        pltpu.async_copy(bufs[p], out_at(b), wsems[p])
        if b + EBUF < BT:
            pltpu.make_async_copy(bufs[p], out_at(b), wsems[p]).wait()
            start_g(b + EBUF, p)
    for b in range(BT - EBUF, BT):
        p = b % EBUF
        pltpu.make_async_copy(bufs[p], out_at(b), wsems[p]).wait()


@functools.partial(
    pl.kernel,
    out_type=jax.ShapeDtypeStruct((NC, NPA, 16), jnp.float32),
    mesh=_mesh,
    scratch_types=[
        pltpu.VMEM((T, CH), jnp.int32),
        pltpu.VMEM((CH, 16), jnp.float32),
        pltpu.VMEM_SHARED((NPA, 16), jnp.float32),
        pltpu.SemaphoreType.DMA,
    ],
)
def _degree(dst_hbm, zeros_hbm, ones_hbm, out_hbm, dst_v, ones_v, acc_sh, sem):
    cid = lax.axis_index("c")
    sid = lax.axis_index("s")
    wid = sid * NC + cid
    pltpu.sync_copy(zeros_hbm, acc_sh.at[pl.ds(sid * RWA, RWA)])
    pltpu.sync_copy(ones_hbm, ones_v)
    pltpu.sync_copy(dst_hbm.at[pl.ds(wid * T, T)], dst_v)
    plsc.subcore_barrier()

    @pl.loop(0, T)
    def _(t):
        pltpu.async_copy(ones_v, acc_sh.at[dst_v.at[t]], sem, add=True)

    @pl.loop(0, T)
    def _(t):
        pltpu.make_async_copy(ones_v, acc_sh.at[dst_v.at[t]], sem).wait()

    plsc.subcore_barrier()
    pltpu.sync_copy(acc_sh.at[pl.ds(sid * RWA, RWA)],
                    out_hbm.at[cid, pl.ds(sid * RWA, RWA), :])


@functools.partial(
    pl.kernel,
    out_type=jax.ShapeDtypeStruct((NC, NPA, H), jnp.float32),
    mesh=_mesh,
    scratch_types=[
        pltpu.VMEM((PH, CH), jnp.int32),
        pltpu.VMEM((PH, CH), jnp.int32),
        [pltpu.VMEM((CH, H), jnp.float32)] * NBUF,
        pltpu.VMEM_SHARED((NPA, H), jnp.float32),
        [pltpu.SemaphoreType.DMA] * NBUF,
        [pltpu.SemaphoreType.DMA] * NBUF,
    ],
)
def _edge_pass(g_hbm, src_hbm, dst_hbm, zeros_hbm, out_hbm,
               src_v, dst_v, bufs, acc_sh, gsems, ssems):
    cid = lax.axis_index("c")
    sid = lax.axis_index("s")
    pltpu.sync_copy(zeros_hbm, acc_sh.at[pl.ds(sid * RWA, RWA)])

    def wait_gather(t, b):
        pltpu.make_async_copy(g_hbm.at[src_v.at[t]], bufs[b], gsems[b]).wait()

    def start_scatter(t, b):
        pltpu.async_copy(bufs[b], acc_sh.at[dst_v.at[t]], ssems[b], add=True)

    def wait_scatter(t, b):
        pltpu.make_async_copy(bufs[b], acc_sh.at[dst_v.at[t]],
                              ssems[b]).wait()

    def prefetch(wstart):
        pltpu.sync_copy(src_hbm.at[pl.ds(wstart, PH)], src_v)
        pltpu.sync_copy(dst_hbm.at[pl.ds(wstart, PH)], dst_v)
        for b in range(NBUF):
            pltpu.async_copy(g_hbm.at[src_v.at[b]], bufs[b], gsems[b])

    def run(nchunks):
        @pl.loop(0, nchunks // NBUF - 1)
        def _(i):
            t0 = i * NBUF
            for b in range(NBUF):
                wait_gather(t0 + b, b)
                start_scatter(t0 + b, b)
            for b in range(NBUF):
                wait_scatter(t0 + b, b)
                pltpu.async_copy(g_hbm.at[src_v.at[t0 + NBUF + b]], bufs[b],
                                 gsems[b])

        t0 = nchunks - NBUF
        for b in range(NBUF):
            wait_gather(t0 + b, b)
            start_scatter(t0 + b, b)
        for b in range(NBUF):
            wait_scatter(t0 + b, b)

    @pl.when(cid == 1)
    def _():
        prefetch(sid * N0C)

    @pl.when(cid == 0)
    def _():
        prefetch(16 * N0C + sid * N1C)

    plsc.subcore_barrier()

    @pl.when(cid == 1)
    def _():
        run(PH)
        prefetch(sid * N0C + PH)
        run(PH)

    @pl.when(cid == 0)
    def _():
        run(N1C)

    plsc.subcore_barrier()
    pltpu.sync_copy(acc_sh.at[pl.ds(sid * RWA, RWA)],
                    out_hbm.at[cid, pl.ds(sid * RWA, RWA), :])


def _tc_a_body(h0_ref, w1_ref, d0_ref, d1_ref, dinv_ref, g1_ref):
    deg = d0_ref[...] + d1_ref[...] + 1.0
    dinv = lax.rsqrt(deg)
    h1 = jnp.dot(h0_ref[...], w1_ref[...], preferred_element_type=jnp.float32)
    dinv_ref[...] = dinv
    g1_ref[...] = h1 * dinv


def _tc_b_body(s0_ref, s1_ref, g1_ref, dinv_ref,
               w2_ref, b1_ref, g2_ref):
    dinv = dinv_ref[...]
    s = s0_ref[...] + s1_ref[...]
    x = jnp.maximum((s + g1_ref[...]) * dinv + b1_ref[...], 0.0)
    g2_ref[...] = jnp.dot(
        x, w2_ref[...], preferred_element_type=jnp.float32) * dinv


def _tc_c_body(s0_ref, s1_ref, g2_ref, dinv_ref, b2_ref,
               batch_ref, wc_ref, bc_ref, out_ref, pool_acc, cnt_acc):
    i = pl.program_id(0)

    @pl.when(i == 0)
    def _():
        pool_acc[...] = jnp.zeros_like(pool_acc)
        cnt_acc[...] = jnp.zeros_like(cnt_acc)

    s = s0_ref[...] + s1_ref[...]
    x = jnp.maximum(
        (s + g2_ref[...]) * dinv_ref[...] + b2_ref[...], 0.0)
    gids = lax.broadcasted_iota(jnp.int32, (NG, BLK), 0)
    ind = (batch_ref[...].reshape(1, BLK) == gids).astype(jnp.float32)
    pool_acc[...] += jnp.dot(ind, x, preferred_element_type=jnp.float32)
    cnt_acc[...] += jnp.sum(ind, axis=1, keepdims=True)

    @pl.when(i == GRID - 1)
    def _():
        pooled = pool_acc[...] / jnp.maximum(cnt_acc[...], 1.0)
        out_ref[...] = jnp.dot(
            pooled, wc_ref[...], preferred_element_type=jnp.float32) \
            + bc_ref[...]


def _row_spec(width):
    return pl.BlockSpec((BLK, width), lambda i: (i, 0))


def _full_spec(shape):
    return pl.BlockSpec(shape, lambda i: tuple(0 for _ in shape))


def kernel(x, edge_index, batch, emb_table, W1, b1, W2, b2, Wc, bc):
    x = x.astype(jnp.int32)
    src = edge_index[0].astype(jnp.int32)
    dst = edge_index[1].astype(jnp.int32)
    batch = batch.astype(jnp.int32)

    xpad = jnp.concatenate([x, jnp.zeros((NPAD - N,), jnp.int32)])
    xpad = xpad.reshape(NW, BT, BCH)
    srcp = jnp.concatenate(
        [src, jnp.zeros((CHAL * CH - E,), jnp.int32)]).reshape(CHAL, CH)
    dstp = jnp.concatenate(
        [dst, jnp.full((CHAL * CH - E,), NJUNK, jnp.int32)]).reshape(CHAL, CH)

    zrow = jnp.zeros((RWA, H), jnp.float32)
    zdeg = jnp.zeros((RWA, 16), jnp.float32)
    ones16 = jnp.ones((CH, 16), jnp.float32)

    h0 = _emb_gather(emb_table, xpad)[:N]
    degp = _degree(dstp, zdeg, ones16)
    d0 = degp[0, :N, 0:1]
    d1 = degp[1, :N, 0:1]

    dinv, g1 = pl.pallas_call(
        _tc_a_body,
        grid=(GRID,),
        in_specs=[_row_spec(EMB), _full_spec((EMB, H)),
                  _row_spec(1), _row_spec(1)],
        out_specs=[_row_spec(1), _row_spec(H)],
        out_shape=[jax.ShapeDtypeStruct((N, 1), jnp.float32),
                   jax.ShapeDtypeStruct((N, H), jnp.float32)],
        compiler_params=pltpu.CompilerParams(
            dimension_semantics=("parallel",)),
    )(h0, W1, d0, d1)

    s1p = _edge_pass(g1, srcp, dstp, zrow)

    g2 = pl.pallas_call(
        _tc_b_body,
        grid=(GRID,),
        in_specs=[_row_spec(H), _row_spec(H), _row_spec(H), _row_spec(1),
                  _full_spec((H, H)), _full_spec((1, H))],
        out_specs=_row_spec(H),
        out_shape=jax.ShapeDtypeStruct((N, H), jnp.float32),
        compiler_params=pltpu.CompilerParams(
            dimension_semantics=("parallel",)),
    )(s1p[0, :N], s1p[1, :N], g1, dinv, W2, b1.reshape(1, H))

    s2p = _edge_pass(g2, srcp, dstp, zrow)

    out = pl.pallas_call(
        _tc_c_body,
        grid=(GRID,),
        in_specs=[_row_spec(H), _row_spec(H), _row_spec(H), _row_spec(1),
                  _full_spec((1, H)),
                  pl.BlockSpec((1, 1, BLK), lambda i: (i, 0, 0)),
                  _full_spec((H, 1)), _full_spec((1, 1))],
        out_specs=_full_spec((NG, 1)),
        out_shape=jax.ShapeDtypeStruct((NG, 1), jnp.float32),
        scratch_shapes=[pltpu.VMEM((NG, H), jnp.float32),
                        pltpu.VMEM((NG, 1), jnp.float32)],
        compiler_params=pltpu.CompilerParams(
            dimension_semantics=("arbitrary",)),
    )(s2p[0, :N], s2p[1, :N], g2, dinv, b2.reshape(1, H),
      batch.reshape(GRID, 1, BLK), Wc, bc.reshape(1, 1))

    return out
